# Initial kernel scaffold; baseline (speedup 1.0000x reference)
import functools

import jax
import jax.numpy as jnp
from jax import lax
from jax.experimental import pallas as pl
from jax.experimental.pallas import tpu as pltpu
from jax.experimental.pallas import tpu_sc as plsc

ROWS = 64
COLS = 32768
TOPK = 512
LANES = 16
CHUNKS = COLS // LANES

_T0 = 1.8
_MIN_I32 = jnp.int32(-(2**31))
_HI_KEY = jnp.int32(0x7F800000)


def _keys_of(v):
    bits = lax.bitcast_convert_type(v, jnp.int32)
    sgn = lax.shift_right_arithmetic(bits, 31)
    flip = lax.shift_right_logical(sgn, 1)
    return lax.bitwise_xor(bits, flip)


def _kernel_body(x_hbm, o_hbm, buf0, buf1, keys, si0, si1, so0, so1):
    cid = lax.axis_index("c")
    sid = lax.axis_index("s")
    wid = sid * 2 + cid
    r0 = wid * 2

    cp_in0 = pltpu.async_copy(x_hbm.at[r0], buf0, si0)
    cp_in1 = pltpu.async_copy(x_hbm.at[r0 + 1], buf1, si1)

    def compress(buf, t0):
        def it(i, ptr):
            v = buf[pl.ds(i * LANES, LANES)]
            m = v > t0
            plsc.store_compressed(keys.at[pl.ds(ptr, LANES)], _keys_of(v), m)
            pc = plsc.all_reduce_population_count(m)
            return ptr + jnp.max(pc)

        return lax.fori_loop(0, CHUNKS, it, jnp.int32(0))

    def process(buf, row, sem_out):
        c0 = compress(buf, jnp.float32(_T0))
        c0 = lax.cond(
            c0 < TOPK,
            lambda: compress(buf, jnp.float32(float("-inf"))),
            lambda: c0,
        )
        keys[pl.ds(c0, LANES)] = jnp.full((LANES,), _MIN_I32, jnp.int32)
        nch = (c0 + LANES - 1) // LANES

        def count_ge(m):
            def it(j, acc):
                kv = keys[pl.ds(j * LANES, LANES)]
                return acc + jnp.where(kv >= m, 1, 0).astype(jnp.int32)

            acc = lax.fori_loop(0, nch, it, jnp.zeros((LANES,), jnp.int32))
            return jnp.sum(acc)

        def bs_it(_, lohi):
            lo, hi = lohi
            mid = (
                lax.shift_right_arithmetic(lo, 1)
                + lax.shift_right_arithmetic(hi, 1)
                + (lo & hi & 1)
            )
            big = count_ge(mid) >= TOPK
            return (
                jnp.where(big, mid, lo),
                jnp.where(big, hi, mid),
            )

        lo, _ = lax.fori_loop(0, 32, bs_it, (_MIN_I32 + 1, _HI_KEY))
        kth = lo

        @pl.loop(0, CHUNKS)
        def _(i):
            sl = pl.ds(i * LANES, LANES)
            v = buf[sl]
            buf[sl] = jnp.where(_keys_of(v) >= kth, v, jnp.float32(0.0))

        return pltpu.async_copy(buf, o_hbm.at[row], sem_out)

    cp_in0.wait()
    cp_out0 = process(buf0, r0, so0)
    cp_in1.wait()
    cp_out1 = process(buf1, r0 + 1, so1)
    cp_out0.wait()
    cp_out1.wait()


def kernel(x):
    mesh = plsc.VectorSubcoreMesh(core_axis_name="c", subcore_axis_name="s")
    run = pl.kernel(
        _kernel_body,
        out_type=jax.ShapeDtypeStruct((ROWS, COLS), jnp.float32),
        mesh=mesh,
        scratch_types=[
            pltpu.VMEM((COLS,), jnp.float32),
            pltpu.VMEM((COLS,), jnp.float32),
            pltpu.VMEM((COLS + 2 * LANES,), jnp.int32),
            pltpu.SemaphoreType.DMA,
            pltpu.SemaphoreType.DMA,
            pltpu.SemaphoreType.DMA,
            pltpu.SemaphoreType.DMA,
        ],
    )
    return run(x)


# speedup vs baseline: 11.3891x; 11.3891x over previous
import dataclasses
import functools

import jax
import jax.numpy as jnp
from jax import lax
from jax.experimental import pallas as pl
from jax.experimental.pallas import tpu as pltpu
from jax.experimental.pallas import tpu_sc as plsc

ROWS = 64
COLS = 32768
TOPK = 512
LANES = 16
CHUNKS = COLS // LANES

_T0 = 1.8
_MIN_I32 = -(2**31)
_HI_KEY = 0x7F800000


def _keys_of(v):
    bits = lax.bitcast_convert_type(v, jnp.int32)
    sgn = lax.shift_right_arithmetic(bits, 31)
    flip = lax.shift_right_logical(sgn, 1)
    return lax.bitwise_xor(bits, flip)


def _kernel_body(x_hbm, o_hbm, buf0, buf1, keys, si0, si1, so0, so1):
    cid = lax.axis_index("c")
    sid = lax.axis_index("s")
    wid = sid * 2 + cid
    r0 = wid * 2

    cp_in0 = pltpu.async_copy(x_hbm.at[r0], buf0, si0)
    cp_in1 = pltpu.async_copy(x_hbm.at[r0 + 1], buf1, si1)

    def compress(buf, t0):
        def it(i, ptr):
            v = buf[pl.ds(i * LANES, LANES)]
            m = v > t0
            plsc.store_compressed(keys.at[pl.ds(ptr, LANES)], _keys_of(v), mask=m)
            pc = plsc.all_reduce_population_count(m)
            return ptr + jnp.max(pc)

        return lax.fori_loop(0, CHUNKS, it, jnp.int32(0))

    def process(buf, row, sem_out):
        c0 = compress(buf, jnp.float32(_T0))
        c0 = lax.cond(
            c0 < TOPK,
            lambda: compress(buf, jnp.float32(float("-inf"))),
            lambda: c0,
        )
        keys[pl.ds(c0, LANES)] = jnp.full((LANES,), jnp.int32(_MIN_I32))
        nch = (c0 + LANES - 1) // LANES

        def count_ge(m):
            def it(j, acc):
                kv = keys[pl.ds(j * LANES, LANES)]
                return acc + jnp.where(kv >= m, 1, 0).astype(jnp.int32)

            acc = lax.fori_loop(0, nch, it, jnp.zeros((LANES,), jnp.int32))
            return jnp.sum(acc)

        def bs_it(_, lohi):
            lo, hi = lohi
            mid = (
                lax.shift_right_arithmetic(lo, 1)
                + lax.shift_right_arithmetic(hi, 1)
                + (lo & hi & 1)
            )
            big = count_ge(mid) >= TOPK
            return (
                jnp.where(big, mid, lo),
                jnp.where(big, hi, mid),
            )

        lo, _ = lax.fori_loop(
            0, 32, bs_it, (jnp.int32(_MIN_I32 + 1), jnp.int32(_HI_KEY))
        )
        kth = lo

        @pl.loop(0, CHUNKS)
        def _(i):
            sl = pl.ds(i * LANES, LANES)
            v = buf[sl]
            buf[sl] = jnp.where(_keys_of(v) >= kth, v, jnp.float32(0.0))

        return pltpu.async_copy(buf, o_hbm.at[row], sem_out)

    cp_in0.wait()
    cp_out0 = process(buf0, r0, so0)
    cp_in1.wait()
    cp_out1 = process(buf1, r0 + 1, so1)
    cp_out0.wait()
    cp_out1.wait()


def kernel(x):
    mesh = plsc.VectorSubcoreMesh(core_axis_name="c", subcore_axis_name="s")
    cp = pltpu.CompilerParams()
    if "needs_layout_passes" in pltpu.CompilerParams.__dataclass_fields__:
        cp = dataclasses.replace(cp, needs_layout_passes=False)
    run = pl.kernel(
        _kernel_body,
        out_type=jax.ShapeDtypeStruct((ROWS, COLS), jnp.float32),
        mesh=mesh,
        compiler_params=cp,
        scratch_types=[
            pltpu.VMEM((COLS,), jnp.float32),
            pltpu.VMEM((COLS,), jnp.float32),
            pltpu.VMEM((COLS + 2 * LANES,), jnp.int32),
            pltpu.SemaphoreType.DMA,
            pltpu.SemaphoreType.DMA,
            pltpu.SemaphoreType.DMA,
            pltpu.SemaphoreType.DMA,
        ],
    )
    return run(x)


# speedup vs baseline: 19.1102x; 1.6779x over previous
import dataclasses
import functools

import jax
import jax.numpy as jnp
from jax import lax
from jax.experimental import pallas as pl
from jax.experimental.pallas import tpu as pltpu
from jax.experimental.pallas import tpu_sc as plsc

ROWS = 64
COLS = 32768
TOPK = 512
LANES = 16
CHUNKS = COLS // LANES
GROUPS = CHUNKS // 4

_T0 = 1.8
_T0_BITS = 0x3FE66666
_MIN_I32 = -(2**31)
_HI_KEY = 0x7F800000


def _keys_of_bits(bits):
    sgn = lax.shift_right_arithmetic(bits, 31)
    flip = lax.shift_right_logical(sgn, 1)
    return lax.bitwise_xor(bits, flip)


def _kernel_body(x_hbm, o_hbm, buf0, buf1, keys, si0, si1, so0, so1):
    cid = lax.axis_index("c")
    sid = lax.axis_index("s")
    wid = sid * 2 + cid
    r0 = wid * 2

    cp_in0 = pltpu.async_copy(x_hbm.at[r0], buf0, si0)
    cp_in1 = pltpu.async_copy(x_hbm.at[r0 + 1], buf1, si1)

    def find_kth(ngroups, lo0, hi0):
        def cond(c):
            lo, hi = c
            return hi - lo > 1

        def body(c):
            lo, hi = c
            mid = (
                lax.shift_right_arithmetic(lo, 1)
                + lax.shift_right_arithmetic(hi, 1)
                + (lo & hi & 1)
            )

            def cit(j, acc):
                a0, a1 = acc
                b = j * 64
                k0 = keys[pl.ds(b, LANES)]
                k1 = keys[pl.ds(b + 16, LANES)]
                k2 = keys[pl.ds(b + 32, LANES)]
                k3 = keys[pl.ds(b + 48, LANES)]
                one = jnp.int32(1)
                zero = jnp.int32(0)
                a0 = a0 + jnp.where(k0 >= mid, one, zero)
                a1 = a1 + jnp.where(k1 >= mid, one, zero)
                a0 = a0 + jnp.where(k2 >= mid, one, zero)
                a1 = a1 + jnp.where(k3 >= mid, one, zero)
                return (a0, a1)

            z = jnp.zeros((LANES,), jnp.int32)
            a0, a1 = lax.fori_loop(0, ngroups, cit, (z, z))
            big = jnp.sum(a0 + a1) >= TOPK
            return (jnp.where(big, mid, lo), jnp.where(big, hi, mid))

        lo, _ = lax.while_loop(cond, body, (lo0, hi0))
        return lo

    def process(buf, row, sem_out):
        t0 = jnp.float32(_T0)

        def comp_it(i, carry):
            ptr, mx = carry
            base = i * 64
            for u in range(4):
                v = buf[pl.ds(base + u * LANES, LANES)]
                m = v > t0
                kb = lax.bitcast_convert_type(v, jnp.int32)
                plsc.store_compressed(keys.at[pl.ds(ptr, LANES)], kb, mask=m)
                pc = plsc.all_reduce_population_count(m)
                ptr = ptr + pc[0]
                mx = jnp.maximum(mx, v)
            return (ptr, mx)

        c0, mxv = lax.fori_loop(
            0,
            GROUPS,
            comp_it,
            (jnp.int32(0), jnp.full((LANES,), jnp.float32(_T0))),
        )

        def fast_fill():
            zpad = jnp.zeros((LANES,), jnp.int32)
            for u in range(4):
                keys[pl.ds(c0 + u * LANES, LANES)] = zpad
            ng = (c0 + 63) >> 6
            hi0 = lax.bitcast_convert_type(jnp.max(mxv), jnp.int32) + 1
            kth = find_kth(ng, jnp.int32(_T0_BITS), hi0)

            @pl.loop(0, GROUPS)
            def _(i):
                base = i * 64
                for u in range(4):
                    sl = pl.ds(base + u * LANES, LANES)
                    v = buf[sl]
                    bits = lax.bitcast_convert_type(v, jnp.int32)
                    buf[sl] = jnp.where(bits >= kth, v, jnp.float32(0.0))

        def fallback_fill():
            @pl.loop(0, GROUPS)
            def _(i):
                base = i * 64
                for u in range(4):
                    sl = pl.ds(base + u * LANES, LANES)
                    bits = lax.bitcast_convert_type(buf[sl], jnp.int32)
                    keys[sl] = _keys_of_bits(bits)

            kth = find_kth(GROUPS, jnp.int32(_MIN_I32 + 1), jnp.int32(_HI_KEY))

            @pl.loop(0, GROUPS)
            def _(i):
                base = i * 64
                for u in range(4):
                    sl = pl.ds(base + u * LANES, LANES)
                    v = buf[sl]
                    bits = lax.bitcast_convert_type(v, jnp.int32)
                    buf[sl] = jnp.where(
                        _keys_of_bits(bits) >= kth, v, jnp.float32(0.0)
                    )

        lax.cond(c0 < TOPK, fallback_fill, fast_fill)
        return pltpu.async_copy(buf, o_hbm.at[row], sem_out)

    cp_in0.wait()
    cp_out0 = process(buf0, r0, so0)
    cp_in1.wait()
    cp_out1 = process(buf1, r0 + 1, so1)
    cp_out0.wait()
    cp_out1.wait()


def kernel(x):
    mesh = plsc.VectorSubcoreMesh(core_axis_name="c", subcore_axis_name="s")
    cp = pltpu.CompilerParams()
    if "needs_layout_passes" in pltpu.CompilerParams.__dataclass_fields__:
        cp = dataclasses.replace(cp, needs_layout_passes=False)
    run = pl.kernel(
        _kernel_body,
        out_type=jax.ShapeDtypeStruct((ROWS, COLS), jnp.float32),
        mesh=mesh,
        compiler_params=cp,
        scratch_types=[
            pltpu.VMEM((COLS,), jnp.float32),
            pltpu.VMEM((COLS,), jnp.float32),
            pltpu.VMEM((COLS + 4 * LANES,), jnp.int32),
            pltpu.SemaphoreType.DMA,
            pltpu.SemaphoreType.DMA,
            pltpu.SemaphoreType.DMA,
            pltpu.SemaphoreType.DMA,
        ],
    )
    return run(x)
